# Initial kernel scaffold; baseline (speedup 1.0000x reference)
import jax
import jax.numpy as jnp
from jax.experimental import pallas as pl


def kernel(feat, edge_index, beta):
    raise NotImplementedError("write your pallas kernel here")


# speedup vs baseline: 5.8551x; 5.8551x over previous
import jax
import jax.numpy as jnp
from jax import lax
from jax.experimental import pallas as pl
from jax.experimental.pallas import tpu as pltpu
from jax.experimental.pallas import tpu_sc as plsc

N_NODES = 10000
N_EDGES = 320000
D_FEAT = 128
D_ACC = D_FEAT + 16

NUM_CORES = 2
NUM_SUBCORES = 16
NUM_TILES = NUM_CORES * NUM_SUBCORES
E_B = 64
N_BATCHES = N_EDGES // E_B
NB_FLOOR = N_BATCHES // NUM_TILES
NB_REM = N_BATCHES % NUM_TILES
N_PAIRS = (NB_FLOOR + (1 if NB_REM else 0) + 1) // 2
ROWS_PER_TILE = N_NODES // NUM_SUBCORES
ROWS_PER_STAGE = 25


def _prep_body(beta_ref, feat_ref, aug_ref, dstm_ref):
    f = feat_ref[...]
    nrm = jnp.sqrt(jnp.sum(f * f, axis=1, keepdims=True))
    nh = f / jnp.maximum(nrm, 1e-12)
    aug_ref[:, :D_FEAT] = nh
    aug_ref[:, D_FEAT:] = nrm * (lax.broadcasted_iota(
        jnp.int32, (1, D_ACC - D_FEAT), 1) == 0).astype(jnp.float32)
    dstm_ref[...] = nh * beta_ref[0]


def _prep(feat, beta):
    return pl.pallas_call(
        _prep_body,
        out_shape=(
            jax.ShapeDtypeStruct((N_NODES, D_ACC), jnp.float32),
            jax.ShapeDtypeStruct((N_NODES, D_FEAT), jnp.float32),
        ),
        in_specs=[
            pl.BlockSpec(memory_space=pltpu.SMEM),
            pl.BlockSpec(memory_space=pltpu.VMEM),
        ],
    )(beta, feat)


def _sc_body(aug_hbm, dstm_hbm, edges_hbm, out_hbm,
             src_idx0, src_idx1, dst_idx0, dst_idx1,
             src_v0, src_v1, dst_v0, dst_v1,
             acc_sh, sem_i, sem_g0, sem_g1, sem_s0, sem_s1):
    c = lax.axis_index("c")
    s = lax.axis_index("s")
    wid = c * NUM_SUBCORES + s
    nb = NB_FLOOR + (wid < NB_REM).astype(jnp.int32)

    src_idx = [src_idx0, src_idx1]
    dst_idx = [dst_idx0, dst_idx1]
    src_v = [src_v0, src_v1]
    dst_v = [dst_v0, dst_v1]
    sem_g = [sem_g0, sem_g1]
    sem_s = [sem_s0, sem_s1]

    z16 = jnp.zeros((16,), jnp.float32)

    def _zero_row(r, carry):
        for k in range(D_ACC // 16):
            src_v0[r, pl.ds(k * 16, 16)] = z16
        return carry

    lax.fori_loop(0, ROWS_PER_STAGE, _zero_row, 0)

    def _zero_chunk(j, carry):
        pltpu.sync_copy(
            src_v0.at[pl.ds(0, ROWS_PER_STAGE)],
            acc_sh.at[pl.ds(s * ROWS_PER_TILE + j * ROWS_PER_STAGE,
                            ROWS_PER_STAGE)])
        return carry

    lax.fori_loop(0, ROWS_PER_TILE // ROWS_PER_STAGE, _zero_chunk, 0)
    plsc.subcore_barrier()

    lane_ids = lax.iota(jnp.int32, 16)
    lane0 = jnp.where(lane_ids == 0, 1.0, 0.0).astype(jnp.float32)
    norm_col = jnp.full((16,), D_FEAT, jnp.int32)

    def _edge_off(b):
        return (b * NUM_TILES + wid) * E_B

    def _fetch_idx(buf, b):
        off = _edge_off(b)
        cpa = pltpu.async_copy(edges_hbm.at[0, pl.ds(off, E_B)],
                               src_idx[buf], sem_i)
        cpb = pltpu.async_copy(edges_hbm.at[1, pl.ds(off, E_B)],
                               dst_idx[buf], sem_i)
        cpa.wait()
        cpb.wait()

    def _start_gathers(buf):
        pltpu.async_copy(aug_hbm.at[src_idx[buf]], src_v[buf], sem_g[buf])
        pltpu.async_copy(dstm_hbm.at[dst_idx[buf]], dst_v[buf], sem_g[buf])

    def _wait_gathers(buf):
        pltpu.make_async_copy(aug_hbm.at[src_idx[buf]], src_v[buf],
                              sem_g[buf]).wait()
        pltpu.make_async_copy(dstm_hbm.at[dst_idx[buf]], dst_v[buf],
                              sem_g[buf]).wait()

    def _start_scatter(buf):
        pltpu.async_copy(src_v[buf], acc_sh.at[dst_idx[buf]], sem_s[buf],
                         add=True)

    def _wait_scatter(buf):
        pltpu.make_async_copy(src_v[buf], acc_sh.at[dst_idx[buf]],
                              sem_s[buf]).wait()

    def _compute(buf):
        def _group(g, carry):
            base = g * 16
            rows = lane_ids + base

            def _dstep(d, acc):
                cols = jnp.full((16,), d, jnp.int32)
                va = plsc.load_gather(src_v[buf], [rows, cols])
                vb = plsc.load_gather(dst_v[buf], [rows, cols])
                return acc + va * vb

            dot = lax.fori_loop(0, D_FEAT, _dstep,
                                jnp.zeros((16,), jnp.float32), unroll=8)
            w = jnp.exp(dot)
            vn = plsc.load_gather(src_v[buf], [rows, norm_col])
            scale = w * vn

            for l in range(16):
                scv = jnp.full((16,), scale[l], jnp.float32)
                wvv = jnp.full((16,), w[l], jnp.float32)
                e = base + l
                for k in range(D_FEAT // 16):
                    src_v[buf][e, pl.ds(k * 16, 16)] = (
                        src_v[buf][e, pl.ds(k * 16, 16)] * scv)
                src_v[buf][e, pl.ds(D_FEAT, 16)] = lane0 * wvv
            return carry

        lax.fori_loop(0, E_B // 16, _group, 0)

    _fetch_idx(0, jnp.int32(0))
    _start_gathers(0)

    def _pair(p, carry):
        for cur in (0, 1):
            b = 2 * p + cur
            nxt = 1 - cur

            @pl.when(b < nb)
            def _():
                _wait_gathers(cur)

                @pl.when(b >= 1)
                def _():
                    _wait_scatter(nxt)

                @pl.when(b + 1 < nb)
                def _():
                    _fetch_idx(nxt, b + 1)
                    _start_gathers(nxt)

                _compute(cur)
                _start_scatter(cur)
        return carry

    lax.fori_loop(0, N_PAIRS, _pair, 0)

    @pl.when(nb % 2 == 1)
    def _():
        _wait_scatter(0)

    @pl.when(nb % 2 == 0)
    def _():
        _wait_scatter(1)

    plsc.subcore_barrier()

    def _out_chunk(j, carry):
        r0 = s * ROWS_PER_TILE + j * ROWS_PER_STAGE
        pltpu.sync_copy(acc_sh.at[pl.ds(r0, ROWS_PER_STAGE)],
                        src_v0.at[pl.ds(0, ROWS_PER_STAGE)])
        pltpu.sync_copy(src_v0.at[pl.ds(0, ROWS_PER_STAGE)],
                        out_hbm.at[c, pl.ds(r0, ROWS_PER_STAGE)])
        return carry

    lax.fori_loop(0, ROWS_PER_TILE // ROWS_PER_STAGE, _out_chunk, 0)


def _sc_edge(aug, dstm, edges):
    mesh = plsc.VectorSubcoreMesh(core_axis_name="c", subcore_axis_name="s",
                                  num_cores=NUM_CORES,
                                  num_subcores=NUM_SUBCORES)
    fn = pl.kernel(
        _sc_body,
        out_type=jax.ShapeDtypeStruct((NUM_CORES, N_NODES, D_ACC),
                                      jnp.float32),
        mesh=mesh,
        scratch_types=[
            pltpu.VMEM((E_B,), jnp.int32),
            pltpu.VMEM((E_B,), jnp.int32),
            pltpu.VMEM((E_B,), jnp.int32),
            pltpu.VMEM((E_B,), jnp.int32),
            pltpu.VMEM((E_B, D_ACC), jnp.float32),
            pltpu.VMEM((E_B, D_ACC), jnp.float32),
            pltpu.VMEM((E_B, D_FEAT), jnp.float32),
            pltpu.VMEM((E_B, D_FEAT), jnp.float32),
            pltpu.VMEM_SHARED((N_NODES, D_ACC), jnp.float32),
            pltpu.SemaphoreType.DMA,
            pltpu.SemaphoreType.DMA,
            pltpu.SemaphoreType.DMA,
            pltpu.SemaphoreType.DMA,
            pltpu.SemaphoreType.DMA,
        ],
        compiler_params=pltpu.CompilerParams(use_tc_tiling_on_sc=False,
                                             needs_layout_passes=False),
    )
    return fn(aug, dstm, edges)


def _fin_body(p_ref, o_ref):
    p = p_ref[...]
    tot = p[0] + p[1]
    num = tot[:, :D_FEAT]
    den = tot[:, D_FEAT:D_FEAT + 1]
    o_ref[...] = num / jnp.maximum(den, 1e-20)


def _fin(partial):
    return pl.pallas_call(
        _fin_body,
        out_shape=jax.ShapeDtypeStruct((N_NODES, D_FEAT), jnp.float32),
        grid=(5,),
        in_specs=[pl.BlockSpec((NUM_CORES, N_NODES // 5, D_ACC),
                               lambda i: (0, i, 0))],
        out_specs=pl.BlockSpec((N_NODES // 5, D_FEAT), lambda i: (i, 0)),
    )(partial)


def kernel(feat, edge_index, beta):
    feat = feat.astype(jnp.float32)
    edges = edge_index.astype(jnp.int32)
    beta = beta.astype(jnp.float32)
    aug, dstm = _prep(feat, beta)
    partial = _sc_edge(aug, dstm, edges)
    return _fin(partial)


# speedup vs baseline: 12.8112x; 2.1880x over previous
import jax
import jax.numpy as jnp
from jax import lax
from jax.experimental import pallas as pl
from jax.experimental.pallas import tpu as pltpu
from jax.experimental.pallas import tpu_sc as plsc

N_NODES = 10000
N_EDGES = 320000
D_FEAT = 128
D_ACC = D_FEAT + 16

NUM_CORES = 2
NUM_SUBCORES = 16
NUM_TILES = NUM_CORES * NUM_SUBCORES
E_B = 64
N_BATCHES = N_EDGES // E_B
NB_FLOOR = N_BATCHES // NUM_TILES
NB_REM = N_BATCHES % NUM_TILES
N_PAIRS = (NB_FLOOR + (1 if NB_REM else 0) + 1) // 2
ROWS_PER_TILE = N_NODES // NUM_SUBCORES
ROWS_PER_STAGE = 25


def _prep_body(beta_ref, feat_ref, aug_ref, dstm_ref):
    f = feat_ref[...]
    nrm = jnp.sqrt(jnp.sum(f * f, axis=1, keepdims=True))
    nh = f / jnp.maximum(nrm, 1e-12)
    aug_ref[:, :D_FEAT] = nh
    aug_ref[:, D_FEAT:] = nrm * (lax.broadcasted_iota(
        jnp.int32, (1, D_ACC - D_FEAT), 1) == 0).astype(jnp.float32)
    dstm_ref[...] = nh * beta_ref[0]


def _prep(feat, beta):
    return pl.pallas_call(
        _prep_body,
        out_shape=(
            jax.ShapeDtypeStruct((N_NODES, D_ACC), jnp.float32),
            jax.ShapeDtypeStruct((N_NODES, D_FEAT), jnp.float32),
        ),
        in_specs=[
            pl.BlockSpec(memory_space=pltpu.SMEM),
            pl.BlockSpec(memory_space=pltpu.VMEM),
        ],
    )(beta, feat)


def _sc_body(aug_hbm, dstm_hbm, edges_hbm, out_hbm,
             src_idx0, src_idx1, dst_idx0, dst_idx1,
             src_v0, src_v1, dst_v0, dst_v1,
             dots_v, acc_sh, sem_i, sem_g0, sem_g1, sem_s0, sem_s1):
    c = lax.axis_index("c")
    s = lax.axis_index("s")
    wid = c * NUM_SUBCORES + s
    nb = NB_FLOOR + (wid < NB_REM).astype(jnp.int32)

    src_idx = [src_idx0, src_idx1]
    dst_idx = [dst_idx0, dst_idx1]
    src_v = [src_v0, src_v1]
    dst_v = [dst_v0, dst_v1]
    sem_g = [sem_g0, sem_g1]
    sem_s = [sem_s0, sem_s1]

    z16 = jnp.zeros((16,), jnp.float32)

    def _zero_row(r, carry):
        for k in range(D_ACC // 16):
            src_v0[r, pl.ds(k * 16, 16)] = z16
        return carry

    lax.fori_loop(0, ROWS_PER_STAGE, _zero_row, 0)

    def _zero_chunk(j, carry):
        pltpu.sync_copy(
            src_v0.at[pl.ds(0, ROWS_PER_STAGE)],
            acc_sh.at[pl.ds(s * ROWS_PER_TILE + j * ROWS_PER_STAGE,
                            ROWS_PER_STAGE)])
        return carry

    lax.fori_loop(0, ROWS_PER_TILE // ROWS_PER_STAGE, _zero_chunk, 0)
    plsc.subcore_barrier()

    lane_ids = lax.iota(jnp.int32, 16)
    lane0 = jnp.where(lane_ids == 0, 1.0, 0.0).astype(jnp.float32)
    norm_col = jnp.full((16,), D_FEAT, jnp.int32)
    last_col = jnp.full((16,), 15, jnp.int32)

    def _edge_off(b):
        return (b * NUM_TILES + wid) * E_B

    def _fetch_idx(buf, b):
        off = _edge_off(b)
        cpa = pltpu.async_copy(edges_hbm.at[0, pl.ds(off, E_B)],
                               src_idx[buf], sem_i)
        cpb = pltpu.async_copy(edges_hbm.at[1, pl.ds(off, E_B)],
                               dst_idx[buf], sem_i)
        cpa.wait()
        cpb.wait()

    def _start_gathers(buf):
        pltpu.async_copy(aug_hbm.at[src_idx[buf]], src_v[buf], sem_g[buf])
        pltpu.async_copy(dstm_hbm.at[dst_idx[buf]], dst_v[buf], sem_g[buf])

    def _wait_gathers(buf):
        pltpu.make_async_copy(aug_hbm.at[src_idx[buf]], src_v[buf],
                              sem_g[buf]).wait()
        pltpu.make_async_copy(dstm_hbm.at[dst_idx[buf]], dst_v[buf],
                              sem_g[buf]).wait()

    def _start_scatter(buf):
        pltpu.async_copy(src_v[buf], acc_sh.at[dst_idx[buf]], sem_s[buf],
                         add=True)

    def _wait_scatter(buf):
        pltpu.make_async_copy(src_v[buf], acc_sh.at[dst_idx[buf]],
                              sem_s[buf]).wait()

    def _compute(buf):
        def _group(g, carry):
            base = g * 16
            rows = lane_ids + base
            for l in range(16):
                e = base + l
                acc = (src_v[buf][e, pl.ds(0, 16)] *
                       dst_v[buf][e, pl.ds(0, 16)])
                for k in range(1, D_FEAT // 16):
                    acc = acc + (src_v[buf][e, pl.ds(k * 16, 16)] *
                                 dst_v[buf][e, pl.ds(k * 16, 16)])
                dots_v[l, :] = plsc.cumsum(acc)
            dot = plsc.load_gather(dots_v, [lane_ids, last_col])
            w = jnp.exp(dot)
            vn = plsc.load_gather(src_v[buf], [rows, norm_col])
            scale = w * vn

            for l in range(16):
                scv = jnp.full((16,), scale[l], jnp.float32)
                wvv = jnp.full((16,), w[l], jnp.float32)
                e = base + l
                for k in range(D_FEAT // 16):
                    src_v[buf][e, pl.ds(k * 16, 16)] = (
                        src_v[buf][e, pl.ds(k * 16, 16)] * scv)
                src_v[buf][e, pl.ds(D_FEAT, 16)] = lane0 * wvv
            return carry

        lax.fori_loop(0, E_B // 16, _group, 0)

    _fetch_idx(0, jnp.int32(0))
    _start_gathers(0)

    def _pair(p, carry):
        for cur in (0, 1):
            b = 2 * p + cur
            nxt = 1 - cur

            @pl.when(b < nb)
            def _():
                _wait_gathers(cur)

                @pl.when(b >= 1)
                def _():
                    _wait_scatter(nxt)

                @pl.when(b + 1 < nb)
                def _():
                    _fetch_idx(nxt, b + 1)
                    _start_gathers(nxt)

                _compute(cur)
                _start_scatter(cur)
        return carry

    lax.fori_loop(0, N_PAIRS, _pair, 0)

    @pl.when(nb % 2 == 1)
    def _():
        _wait_scatter(0)

    @pl.when(nb % 2 == 0)
    def _():
        _wait_scatter(1)

    plsc.subcore_barrier()

    def _out_chunk(j, carry):
        r0 = s * ROWS_PER_TILE + j * ROWS_PER_STAGE
        pltpu.sync_copy(acc_sh.at[pl.ds(r0, ROWS_PER_STAGE)],
                        src_v0.at[pl.ds(0, ROWS_PER_STAGE)])
        pltpu.sync_copy(src_v0.at[pl.ds(0, ROWS_PER_STAGE)],
                        out_hbm.at[c, pl.ds(r0, ROWS_PER_STAGE)])
        return carry

    lax.fori_loop(0, ROWS_PER_TILE // ROWS_PER_STAGE, _out_chunk, 0)


def _sc_edge(aug, dstm, edges):
    mesh = plsc.VectorSubcoreMesh(core_axis_name="c", subcore_axis_name="s",
                                  num_cores=NUM_CORES,
                                  num_subcores=NUM_SUBCORES)
    fn = pl.kernel(
        _sc_body,
        out_type=jax.ShapeDtypeStruct((NUM_CORES, N_NODES, D_ACC),
                                      jnp.float32),
        mesh=mesh,
        scratch_types=[
            pltpu.VMEM((E_B,), jnp.int32),
            pltpu.VMEM((E_B,), jnp.int32),
            pltpu.VMEM((E_B,), jnp.int32),
            pltpu.VMEM((E_B,), jnp.int32),
            pltpu.VMEM((E_B, D_ACC), jnp.float32),
            pltpu.VMEM((E_B, D_ACC), jnp.float32),
            pltpu.VMEM((E_B, D_FEAT), jnp.float32),
            pltpu.VMEM((E_B, D_FEAT), jnp.float32),
            pltpu.VMEM((16, 16), jnp.float32),
            pltpu.VMEM_SHARED((N_NODES, D_ACC), jnp.float32),
            pltpu.SemaphoreType.DMA,
            pltpu.SemaphoreType.DMA,
            pltpu.SemaphoreType.DMA,
            pltpu.SemaphoreType.DMA,
            pltpu.SemaphoreType.DMA,
        ],
        compiler_params=pltpu.CompilerParams(use_tc_tiling_on_sc=False,
                                             needs_layout_passes=False),
    )
    return fn(aug, dstm, edges)


def _fin_body(p_ref, o_ref):
    p = p_ref[...]
    tot = p[0] + p[1]
    num = tot[:, :D_FEAT]
    den = tot[:, D_FEAT:D_FEAT + 1]
    o_ref[...] = num / jnp.maximum(den, 1e-20)


def _fin(partial):
    return pl.pallas_call(
        _fin_body,
        out_shape=jax.ShapeDtypeStruct((N_NODES, D_FEAT), jnp.float32),
        grid=(5,),
        in_specs=[pl.BlockSpec((NUM_CORES, N_NODES // 5, D_ACC),
                               lambda i: (0, i, 0))],
        out_specs=pl.BlockSpec((N_NODES // 5, D_FEAT), lambda i: (i, 0)),
    )(partial)


def kernel(feat, edge_index, beta):
    feat = feat.astype(jnp.float32)
    edges = edge_index.astype(jnp.int32)
    beta = beta.astype(jnp.float32)
    aug, dstm = _prep(feat, beta)
    partial = _sc_edge(aug, dstm, edges)
    return _fin(partial)


# speedup vs baseline: 14.4121x; 1.1250x over previous
import jax
import jax.numpy as jnp
from jax import lax
from jax.experimental import pallas as pl
from jax.experimental.pallas import tpu as pltpu
from jax.experimental.pallas import tpu_sc as plsc

N_NODES = 10000
N_EDGES = 320000
D_FEAT = 128
D_ACC = D_FEAT + 16

NUM_CORES = 2
NUM_SUBCORES = 16
NUM_TILES = NUM_CORES * NUM_SUBCORES
E_B = 64
T_BATCH = 160
T_EDGES = T_BATCH * E_B
E_PAD = NUM_TILES * T_EDGES
CHUNK_B = 32
CHUNK_E = CHUNK_B * E_B
N_PAIRS = T_BATCH // 2
ROWS_PER_TILE = N_NODES // NUM_SUBCORES
ROWS_PER_STAGE = 25


def _prep_body(beta_ref, feat_ref, aug_ref, dstm_ref):
    f = feat_ref[...]
    nrm = jnp.sqrt(jnp.sum(f * f, axis=1, keepdims=True))
    nh = f / jnp.maximum(nrm, 1e-12)
    aug_ref[:, :D_FEAT] = nh
    aug_ref[:, D_FEAT:] = nrm * (lax.broadcasted_iota(
        jnp.int32, (1, D_ACC - D_FEAT), 1) == 0).astype(jnp.float32)
    dstm_ref[...] = nh * beta_ref[0]


def _prep(feat, beta):
    return pl.pallas_call(
        _prep_body,
        out_shape=(
            jax.ShapeDtypeStruct((N_NODES, D_ACC), jnp.float32),
            jax.ShapeDtypeStruct((N_NODES, D_FEAT), jnp.float32),
        ),
        in_specs=[
            pl.BlockSpec(memory_space=pltpu.SMEM),
            pl.BlockSpec(memory_space=pltpu.VMEM),
        ],
    )(beta, feat)


def _sc_body(aug_hbm, dstm_hbm, edges_hbm, out_hbm,
             src_chunk, dst_chunk, dst_idx0, dst_idx1,
             src_v0, src_v1, dst_v0, dst_v1,
             dots_v, acc_sh, sem_i, sem_g0, sem_g1, sem_s0, sem_s1):
    c = lax.axis_index("c")
    s = lax.axis_index("s")
    wid = c * NUM_SUBCORES + s
    nb = jnp.minimum((N_EDGES - wid * T_EDGES) // E_B, T_BATCH)

    dst_idx = [dst_idx0, dst_idx1]
    src_v = [src_v0, src_v1]
    dst_v = [dst_v0, dst_v1]
    sem_g = [sem_g0, sem_g1]
    sem_s = [sem_s0, sem_s1]

    z16 = jnp.zeros((16,), jnp.float32)

    def _zero_row(r, carry):
        for k in range(D_ACC // 16):
            src_v0[r, pl.ds(k * 16, 16)] = z16
        return carry

    lax.fori_loop(0, ROWS_PER_STAGE, _zero_row, 0)

    def _zero_chunk(j, carry):
        pltpu.sync_copy(
            src_v0.at[pl.ds(0, ROWS_PER_STAGE)],
            acc_sh.at[pl.ds(s * ROWS_PER_TILE + j * ROWS_PER_STAGE,
                            ROWS_PER_STAGE)])
        return carry

    lax.fori_loop(0, ROWS_PER_TILE // ROWS_PER_STAGE, _zero_chunk, 0)
    plsc.subcore_barrier()

    lane_ids = lax.iota(jnp.int32, 16)
    lane0 = jnp.where(lane_ids == 0, 1.0, 0.0).astype(jnp.float32)
    norm_col = jnp.full((16,), D_FEAT, jnp.int32)
    last_col = jnp.full((16,), 15, jnp.int32)

    def _fetch_chunk(cb):
        off = wid * T_EDGES + cb * CHUNK_E
        cpa = pltpu.async_copy(edges_hbm.at[0, pl.ds(off, CHUNK_E)],
                               src_chunk, sem_i)
        cpb = pltpu.async_copy(edges_hbm.at[1, pl.ds(off, CHUNK_E)],
                               dst_chunk, sem_i)
        cpa.wait()
        cpb.wait()

    def _copy_dst_idx(buf, j):
        for k in range(E_B // 16):
            dst_idx[buf][pl.ds(k * 16, 16)] = (
                dst_chunk[pl.ds(j * E_B + k * 16, 16)])

    def _start_gathers(buf, j):
        pltpu.async_copy(aug_hbm.at[src_chunk.at[pl.ds(j * E_B, E_B)]],
                         src_v[buf], sem_g[buf])
        pltpu.async_copy(dstm_hbm.at[dst_idx[buf]], dst_v[buf], sem_g[buf])

    def _wait_gathers(buf, j):
        pltpu.make_async_copy(aug_hbm.at[src_chunk.at[pl.ds(j * E_B, E_B)]],
                              src_v[buf], sem_g[buf]).wait()
        pltpu.make_async_copy(dstm_hbm.at[dst_idx[buf]], dst_v[buf],
                              sem_g[buf]).wait()

    def _start_scatter(buf):
        pltpu.async_copy(src_v[buf], acc_sh.at[dst_idx[buf]], sem_s[buf],
                         add=True)

    def _wait_scatter(buf):
        pltpu.make_async_copy(src_v[buf], acc_sh.at[dst_idx[buf]],
                              sem_s[buf]).wait()

    def _compute(buf):
        def _group(g, carry):
            base = g * 16
            rows = lane_ids + base
            for l in range(16):
                e = base + l
                acc = (src_v[buf][e, pl.ds(0, 16)] *
                       dst_v[buf][e, pl.ds(0, 16)])
                for k in range(1, D_FEAT // 16):
                    acc = acc + (src_v[buf][e, pl.ds(k * 16, 16)] *
                                 dst_v[buf][e, pl.ds(k * 16, 16)])
                dots_v[l, :] = plsc.cumsum(acc)
            dot = plsc.load_gather(dots_v, [lane_ids, last_col])
            w = jnp.exp(dot)
            vn = plsc.load_gather(src_v[buf], [rows, norm_col])
            scale = w * vn

            for l in range(16):
                scv = jnp.full((16,), scale[l], jnp.float32)
                wvv = jnp.full((16,), w[l], jnp.float32)
                e = base + l
                for k in range(D_FEAT // 16):
                    src_v[buf][e, pl.ds(k * 16, 16)] = (
                        src_v[buf][e, pl.ds(k * 16, 16)] * scv)
                src_v[buf][e, pl.ds(D_FEAT, 16)] = lane0 * wvv
            return carry

        lax.fori_loop(0, E_B // 16, _group, 0)

    _fetch_chunk(jnp.int32(0))
    _copy_dst_idx(0, jnp.int32(0))
    _start_gathers(0, jnp.int32(0))

    def _pair(p, carry):
        for cur in (0, 1):
            b = 2 * p + cur
            nxt = 1 - cur

            @pl.when(b < nb)
            def _():
                _wait_gathers(cur, b % CHUNK_B)

                @pl.when(b >= 1)
                def _():
                    _wait_scatter(nxt)

                @pl.when(b + 1 < nb)
                def _():
                    @pl.when((b + 1) % CHUNK_B == 0)
                    def _():
                        _fetch_chunk((b + 1) // CHUNK_B)

                    _copy_dst_idx(nxt, (b + 1) % CHUNK_B)
                    _start_gathers(nxt, (b + 1) % CHUNK_B)

                _compute(cur)
                _start_scatter(cur)
        return carry

    lax.fori_loop(0, N_PAIRS, _pair, 0)

    @pl.when(nb % 2 == 1)
    def _():
        _wait_scatter(0)

    @pl.when(nb % 2 == 0)
    def _():
        _wait_scatter(1)

    plsc.subcore_barrier()

    def _out_chunk(j, carry):
        r0 = s * ROWS_PER_TILE + j * ROWS_PER_STAGE
        pltpu.sync_copy(acc_sh.at[pl.ds(r0, ROWS_PER_STAGE)],
                        src_v0.at[pl.ds(0, ROWS_PER_STAGE)])
        pltpu.sync_copy(src_v0.at[pl.ds(0, ROWS_PER_STAGE)],
                        out_hbm.at[c, pl.ds(r0, ROWS_PER_STAGE)])
        return carry

    lax.fori_loop(0, ROWS_PER_TILE // ROWS_PER_STAGE, _out_chunk, 0)


def _sc_edge(aug, dstm, edges):
    mesh = plsc.VectorSubcoreMesh(core_axis_name="c", subcore_axis_name="s",
                                  num_cores=NUM_CORES,
                                  num_subcores=NUM_SUBCORES)
    fn = pl.kernel(
        _sc_body,
        out_type=jax.ShapeDtypeStruct((NUM_CORES, N_NODES, D_ACC),
                                      jnp.float32),
        mesh=mesh,
        scratch_types=[
            pltpu.VMEM((CHUNK_E,), jnp.int32),
            pltpu.VMEM((CHUNK_E,), jnp.int32),
            pltpu.VMEM((E_B,), jnp.int32),
            pltpu.VMEM((E_B,), jnp.int32),
            pltpu.VMEM((E_B, D_ACC), jnp.float32),
            pltpu.VMEM((E_B, D_ACC), jnp.float32),
            pltpu.VMEM((E_B, D_FEAT), jnp.float32),
            pltpu.VMEM((E_B, D_FEAT), jnp.float32),
            pltpu.VMEM((16, 16), jnp.float32),
            pltpu.VMEM_SHARED((N_NODES, D_ACC), jnp.float32),
            pltpu.SemaphoreType.DMA,
            pltpu.SemaphoreType.DMA,
            pltpu.SemaphoreType.DMA,
            pltpu.SemaphoreType.DMA,
            pltpu.SemaphoreType.DMA,
        ],
        compiler_params=pltpu.CompilerParams(use_tc_tiling_on_sc=False,
                                             needs_layout_passes=False),
    )
    return fn(aug, dstm, edges)


def _fin_body(p_ref, o_ref):
    p = p_ref[...]
    tot = p[0] + p[1]
    num = tot[:, :D_FEAT]
    den = tot[:, D_FEAT:D_FEAT + 1]
    o_ref[...] = num / jnp.maximum(den, 1e-20)


def _fin(partial):
    return pl.pallas_call(
        _fin_body,
        out_shape=jax.ShapeDtypeStruct((N_NODES, D_FEAT), jnp.float32),
        grid=(5,),
        in_specs=[pl.BlockSpec((NUM_CORES, N_NODES // 5, D_ACC),
                               lambda i: (0, i, 0))],
        out_specs=pl.BlockSpec((N_NODES // 5, D_FEAT), lambda i: (i, 0)),
    )(partial)


def kernel(feat, edge_index, beta):
    feat = feat.astype(jnp.float32)
    edges = edge_index.astype(jnp.int32)
    edges = jnp.concatenate(
        [edges, jnp.zeros((2, E_PAD - N_EDGES), jnp.int32)], axis=1)
    beta = beta.astype(jnp.float32)
    aug, dstm = _prep(feat, beta)
    partial = _sc_edge(aug, dstm, edges)
    return _fin(partial)


# speedup vs baseline: 14.4888x; 1.0053x over previous
import jax
import jax.numpy as jnp
import numpy as np
from jax import lax
from jax.experimental import pallas as pl
from jax.experimental.pallas import tpu as pltpu
from jax.experimental.pallas import tpu_sc as plsc

N_NODES = 10000
N_EDGES = 320000
D_FEAT = 128
D_ACC = D_FEAT + 16

NUM_CORES = 2
NUM_SUBCORES = 16
NUM_TILES = NUM_CORES * NUM_SUBCORES
E_B = 64
T_BATCH = 160
T_EDGES = T_BATCH * E_B
E_PAD = NUM_TILES * T_EDGES
CHUNK_B = 32
CHUNK_E = CHUNK_B * E_B
N_PAIRS = T_BATCH // 2
ROWS_PER_TILE = N_NODES // NUM_SUBCORES
ROWS_PER_STAGE = 25

_DST_PERM = np.asarray(
    [32 * k + 16 * h + t for k in range(4) for t in range(16)
     for h in range(2)], dtype=np.int32)


def _prep_body(beta_ref, perm_ref, feat_ref, aug_ref, dstm_ref):
    f = feat_ref[...]
    nrm = jnp.sqrt(jnp.sum(f * f, axis=1, keepdims=True))
    nh = f / jnp.maximum(nrm, 1e-12)
    aug_ref[:, :D_FEAT] = nh
    aug_ref[:, D_FEAT:] = nrm * (lax.broadcasted_iota(
        jnp.int32, (1, D_ACC - D_FEAT), 1) == 0).astype(jnp.float32)
    z = nh * beta_ref[0]
    pidx = jnp.broadcast_to(perm_ref[...][None, :], z.shape)
    dstm_ref[...] = jnp.take_along_axis(z, pidx, axis=1).astype(jnp.bfloat16)


_PREP_R = N_NODES // 10


def _prep(feat, beta):
    return pl.pallas_call(
        _prep_body,
        out_shape=(
            jax.ShapeDtypeStruct((N_NODES, D_ACC), jnp.float32),
            jax.ShapeDtypeStruct((N_NODES, D_FEAT), jnp.bfloat16),
        ),
        grid=(10,),
        in_specs=[
            pl.BlockSpec(memory_space=pltpu.SMEM),
            pl.BlockSpec((128,), lambda i: (0,)),
            pl.BlockSpec((_PREP_R, D_FEAT), lambda i: (i, 0)),
        ],
        out_specs=(
            pl.BlockSpec((_PREP_R, D_ACC), lambda i: (i, 0)),
            pl.BlockSpec((_PREP_R, D_FEAT), lambda i: (i, 0)),
        ),
    )(beta, jnp.asarray(_DST_PERM), feat)


def _sc_body(aug_hbm, dstm_hbm, edges_hbm, out_hbm,
             src_chunk, dst_chunk, dst_idx0, dst_idx1,
             src_v0, src_v1, dst_v0, dst_v1,
             dots_v, acc_sh, sem_i, sem_g0, sem_g1, sem_s0, sem_s1):
    c = lax.axis_index("c")
    s = lax.axis_index("s")
    wid = c * NUM_SUBCORES + s
    nb = jnp.minimum((N_EDGES - wid * T_EDGES) // E_B, T_BATCH)

    dst_idx = [dst_idx0, dst_idx1]
    src_v = [src_v0, src_v1]
    dst_v = [dst_v0, dst_v1]
    sem_g = [sem_g0, sem_g1]
    sem_s = [sem_s0, sem_s1]

    z16 = jnp.zeros((16,), jnp.float32)

    def _zero_row(r, carry):
        for k in range(D_ACC // 16):
            src_v0[r, pl.ds(k * 16, 16)] = z16
        return carry

    lax.fori_loop(0, ROWS_PER_STAGE, _zero_row, 0)

    def _zero_chunk(j, carry):
        pltpu.sync_copy(
            src_v0.at[pl.ds(0, ROWS_PER_STAGE)],
            acc_sh.at[pl.ds(s * ROWS_PER_TILE + j * ROWS_PER_STAGE,
                            ROWS_PER_STAGE)])
        return carry

    lax.fori_loop(0, ROWS_PER_TILE // ROWS_PER_STAGE, _zero_chunk, 0)
    plsc.subcore_barrier()

    lane_ids = lax.iota(jnp.int32, 16)
    lane0 = jnp.where(lane_ids == 0, 1.0, 0.0).astype(jnp.float32)
    norm_col = jnp.full((16,), D_FEAT, jnp.int32)
    last_col = jnp.full((16,), 15, jnp.int32)

    def _fetch_chunk(cb):
        off = wid * T_EDGES + cb * CHUNK_E
        cpa = pltpu.async_copy(edges_hbm.at[0, pl.ds(off, CHUNK_E)],
                               src_chunk, sem_i)
        cpb = pltpu.async_copy(edges_hbm.at[1, pl.ds(off, CHUNK_E)],
                               dst_chunk, sem_i)
        cpa.wait()
        cpb.wait()

    def _copy_dst_idx(buf, j):
        for k in range(E_B // 16):
            dst_idx[buf][pl.ds(k * 16, 16)] = (
                dst_chunk[pl.ds(j * E_B + k * 16, 16)])

    def _start_gathers(buf, j):
        pltpu.async_copy(aug_hbm.at[src_chunk.at[pl.ds(j * E_B, E_B)]],
                         src_v[buf], sem_g[buf])
        pltpu.async_copy(dstm_hbm.at[dst_idx[buf]], dst_v[buf], sem_g[buf])

    def _wait_gathers(buf, j):
        pltpu.make_async_copy(aug_hbm.at[src_chunk.at[pl.ds(j * E_B, E_B)]],
                              src_v[buf], sem_g[buf]).wait()
        pltpu.make_async_copy(dstm_hbm.at[dst_idx[buf]], dst_v[buf],
                              sem_g[buf]).wait()

    def _start_scatter(buf):
        pltpu.async_copy(src_v[buf], acc_sh.at[dst_idx[buf]], sem_s[buf],
                         add=True)

    def _wait_scatter(buf):
        pltpu.make_async_copy(src_v[buf], acc_sh.at[dst_idx[buf]],
                              sem_s[buf]).wait()

    def _compute(buf):
        def _group(g, carry):
            base = g * 16
            rows = lane_ids + base
            for l in range(16):
                e = base + l
                acc = None
                for k in range(D_FEAT // 32):
                    db = dst_v[buf][e, pl.ds(k * 32, 32)]
                    u, v = plsc.unpack(db, format=plsc.PackFormat.INTERLEAVED)
                    term = (u * src_v[buf][e, pl.ds(k * 32, 16)] +
                            v * src_v[buf][e, pl.ds(k * 32 + 16, 16)])
                    acc = term if acc is None else acc + term
                dots_v[l, :] = plsc.cumsum(acc)
            dot = plsc.load_gather(dots_v, [lane_ids, last_col])
            w = jnp.exp(dot)
            vn = plsc.load_gather(src_v[buf], [rows, norm_col])
            scale = w * vn

            for l in range(16):
                scv = jnp.full((16,), scale[l], jnp.float32)
                wvv = jnp.full((16,), w[l], jnp.float32)
                e = base + l
                for k in range(D_FEAT // 16):
                    src_v[buf][e, pl.ds(k * 16, 16)] = (
                        src_v[buf][e, pl.ds(k * 16, 16)] * scv)
                src_v[buf][e, pl.ds(D_FEAT, 16)] = lane0 * wvv
            return carry

        lax.fori_loop(0, E_B // 16, _group, 0)

    _fetch_chunk(jnp.int32(0))
    _copy_dst_idx(0, jnp.int32(0))
    _start_gathers(0, jnp.int32(0))

    def _pair(p, carry):
        for cur in (0, 1):
            b = 2 * p + cur
            nxt = 1 - cur

            @pl.when(b < nb)
            def _():
                _wait_gathers(cur, b % CHUNK_B)

                @pl.when(b >= 1)
                def _():
                    _wait_scatter(nxt)

                @pl.when(b + 1 < nb)
                def _():
                    @pl.when((b + 1) % CHUNK_B == 0)
                    def _():
                        _fetch_chunk((b + 1) // CHUNK_B)

                    _copy_dst_idx(nxt, (b + 1) % CHUNK_B)
                    _start_gathers(nxt, (b + 1) % CHUNK_B)

                _compute(cur)
                _start_scatter(cur)
        return carry

    lax.fori_loop(0, N_PAIRS, _pair, 0)

    @pl.when(nb % 2 == 1)
    def _():
        _wait_scatter(0)

    @pl.when(nb % 2 == 0)
    def _():
        _wait_scatter(1)

    plsc.subcore_barrier()

    def _out_chunk(j, carry):
        r0 = s * ROWS_PER_TILE + j * ROWS_PER_STAGE
        pltpu.sync_copy(acc_sh.at[pl.ds(r0, ROWS_PER_STAGE)],
                        src_v0.at[pl.ds(0, ROWS_PER_STAGE)])
        pltpu.sync_copy(src_v0.at[pl.ds(0, ROWS_PER_STAGE)],
                        out_hbm.at[c, pl.ds(r0, ROWS_PER_STAGE)])
        return carry

    lax.fori_loop(0, ROWS_PER_TILE // ROWS_PER_STAGE, _out_chunk, 0)


def _sc_edge(aug, dstm, edges):
    mesh = plsc.VectorSubcoreMesh(core_axis_name="c", subcore_axis_name="s",
                                  num_cores=NUM_CORES,
                                  num_subcores=NUM_SUBCORES)
    fn = pl.kernel(
        _sc_body,
        out_type=jax.ShapeDtypeStruct((NUM_CORES, N_NODES, D_ACC),
                                      jnp.float32),
        mesh=mesh,
        scratch_types=[
            pltpu.VMEM((CHUNK_E,), jnp.int32),
            pltpu.VMEM((CHUNK_E,), jnp.int32),
            pltpu.VMEM((E_B,), jnp.int32),
            pltpu.VMEM((E_B,), jnp.int32),
            pltpu.VMEM((E_B, D_ACC), jnp.float32),
            pltpu.VMEM((E_B, D_ACC), jnp.float32),
            pltpu.VMEM((E_B, D_FEAT), jnp.bfloat16),
            pltpu.VMEM((E_B, D_FEAT), jnp.bfloat16),
            pltpu.VMEM((16, 16), jnp.float32),
            pltpu.VMEM_SHARED((N_NODES, D_ACC), jnp.float32),
            pltpu.SemaphoreType.DMA,
            pltpu.SemaphoreType.DMA,
            pltpu.SemaphoreType.DMA,
            pltpu.SemaphoreType.DMA,
            pltpu.SemaphoreType.DMA,
        ],
        compiler_params=pltpu.CompilerParams(use_tc_tiling_on_sc=False,
                                             needs_layout_passes=False),
    )
    return fn(aug, dstm, edges)


def _fin_body(p_ref, o_ref):
    p = p_ref[...]
    tot = p[0] + p[1]
    num = tot[:, :D_FEAT]
    den = tot[:, D_FEAT:D_FEAT + 1]
    o_ref[...] = num / jnp.maximum(den, 1e-20)


def _fin(partial):
    return pl.pallas_call(
        _fin_body,
        out_shape=jax.ShapeDtypeStruct((N_NODES, D_FEAT), jnp.float32),
        grid=(5,),
        in_specs=[pl.BlockSpec((NUM_CORES, N_NODES // 5, D_ACC),
                               lambda i: (0, i, 0))],
        out_specs=pl.BlockSpec((N_NODES // 5, D_FEAT), lambda i: (i, 0)),
    )(partial)


def kernel(feat, edge_index, beta):
    feat = feat.astype(jnp.float32)
    edges = edge_index.astype(jnp.int32)
    edges = jnp.concatenate(
        [edges, jnp.zeros((2, E_PAD - N_EDGES), jnp.int32)], axis=1)
    beta = beta.astype(jnp.float32)
    aug, dstm = _prep(feat, beta)
    partial = _sc_edge(aug, dstm, edges)
    return _fin(partial)


# speedup vs baseline: 14.6020x; 1.0078x over previous
import jax
import jax.numpy as jnp
import numpy as np
from jax import lax
from jax.experimental import pallas as pl
from jax.experimental.pallas import tpu as pltpu
from jax.experimental.pallas import tpu_sc as plsc

N_NODES = 10000
N_EDGES = 320000
D_FEAT = 128
D_ACC = D_FEAT + 16

NUM_CORES = 2
NUM_SUBCORES = 16
NUM_TILES = NUM_CORES * NUM_SUBCORES
E_B = 80
T_BATCH = 128
T_EDGES = T_BATCH * E_B
E_PAD = NUM_TILES * T_EDGES
CHUNK_B = 32
CHUNK_E = CHUNK_B * E_B
N_PAIRS = T_BATCH // 2
ROWS_PER_TILE = N_NODES // NUM_SUBCORES
ROWS_PER_STAGE = 25

_DST_PERM = np.asarray(
    [32 * k + 16 * h + t for k in range(4) for t in range(16)
     for h in range(2)], dtype=np.int32)


def _prep_body(beta_ref, perm_ref, feat_ref, aug_ref, dstm_ref):
    f = feat_ref[...]
    nrm = jnp.sqrt(jnp.sum(f * f, axis=1, keepdims=True))
    nh = f / jnp.maximum(nrm, 1e-12)
    aug_ref[:, :D_FEAT] = nh
    aug_ref[:, D_FEAT:] = nrm * (lax.broadcasted_iota(
        jnp.int32, (1, D_ACC - D_FEAT), 1) == 0).astype(jnp.float32)
    z = nh * beta_ref[0]
    pidx = jnp.broadcast_to(perm_ref[...][None, :], z.shape)
    dstm_ref[...] = jnp.take_along_axis(z, pidx, axis=1).astype(jnp.bfloat16)


_PREP_R = N_NODES // 10


def _prep(feat, beta):
    return pl.pallas_call(
        _prep_body,
        out_shape=(
            jax.ShapeDtypeStruct((N_NODES, D_ACC), jnp.float32),
            jax.ShapeDtypeStruct((N_NODES, D_FEAT), jnp.bfloat16),
        ),
        grid=(10,),
        in_specs=[
            pl.BlockSpec(memory_space=pltpu.SMEM),
            pl.BlockSpec((128,), lambda i: (0,)),
            pl.BlockSpec((_PREP_R, D_FEAT), lambda i: (i, 0)),
        ],
        out_specs=(
            pl.BlockSpec((_PREP_R, D_ACC), lambda i: (i, 0)),
            pl.BlockSpec((_PREP_R, D_FEAT), lambda i: (i, 0)),
        ),
    )(beta, jnp.asarray(_DST_PERM), feat)


def _sc_body(aug_hbm, dstm_hbm, edges_hbm, out_hbm,
             src_chunk, dst_chunk, dst_idx0, dst_idx1,
             src_v0, src_v1, dst_v0, dst_v1,
             dots_v, acc_sh, sem_i, sem_g0, sem_g1, sem_s0, sem_s1):
    c = lax.axis_index("c")
    s = lax.axis_index("s")
    wid = c * NUM_SUBCORES + s
    nb = jnp.minimum((N_EDGES - wid * T_EDGES) // E_B, T_BATCH)

    dst_idx = [dst_idx0, dst_idx1]
    src_v = [src_v0, src_v1]
    dst_v = [dst_v0, dst_v1]
    sem_g = [sem_g0, sem_g1]
    sem_s = [sem_s0, sem_s1]

    lane_ids = lax.iota(jnp.int32, 16)
    lane0 = jnp.where(lane_ids == 0, 1.0, 0.0).astype(jnp.float32)
    norm_col = jnp.full((16,), D_FEAT, jnp.int32)
    last_col = jnp.full((16,), 15, jnp.int32)

    def _fetch_chunk(cb):
        off = wid * T_EDGES + cb * CHUNK_E
        cpa = pltpu.async_copy(edges_hbm.at[0, pl.ds(off, CHUNK_E)],
                               src_chunk, sem_i)
        cpb = pltpu.async_copy(edges_hbm.at[1, pl.ds(off, CHUNK_E)],
                               dst_chunk, sem_i)
        cpa.wait()
        cpb.wait()

    def _copy_dst_idx(buf, j):
        for k in range(E_B // 16):
            dst_idx[buf][pl.ds(k * 16, 16)] = (
                dst_chunk[pl.ds(j * E_B + k * 16, 16)])

    def _start_gathers(buf, j):
        pltpu.async_copy(aug_hbm.at[src_chunk.at[pl.ds(j * E_B, E_B)]],
                         src_v[buf], sem_g[buf])
        pltpu.async_copy(dstm_hbm.at[dst_idx[buf]], dst_v[buf], sem_g[buf])

    def _wait_gathers(buf, j):
        pltpu.make_async_copy(aug_hbm.at[src_chunk.at[pl.ds(j * E_B, E_B)]],
                              src_v[buf], sem_g[buf]).wait()
        pltpu.make_async_copy(dstm_hbm.at[dst_idx[buf]], dst_v[buf],
                              sem_g[buf]).wait()

    def _start_scatter(buf):
        pltpu.async_copy(src_v[buf], acc_sh.at[dst_idx[buf]], sem_s[buf],
                         add=True)

    def _wait_scatter(buf):
        pltpu.make_async_copy(src_v[buf], acc_sh.at[dst_idx[buf]],
                              sem_s[buf]).wait()

    def _compute(buf):
        def _group(g, carry):
            base = g * 16
            rows = lane_ids + base
            for l in range(16):
                e = base + l
                acc = None
                for k in range(D_FEAT // 32):
                    db = dst_v[buf][e, pl.ds(k * 32, 32)]
                    u, v = plsc.unpack(db, format=plsc.PackFormat.INTERLEAVED)
                    term = (u * src_v[buf][e, pl.ds(k * 32, 16)] +
                            v * src_v[buf][e, pl.ds(k * 32 + 16, 16)])
                    acc = term if acc is None else acc + term
                dots_v[l, :] = plsc.cumsum(acc)
            dot = plsc.load_gather(dots_v, [lane_ids, last_col])
            w = jnp.exp(dot)
            vn = plsc.load_gather(src_v[buf], [rows, norm_col])
            scale = w * vn

            for l in range(16):
                scv = jnp.full((16,), scale[l], jnp.float32)
                wvv = jnp.full((16,), w[l], jnp.float32)
                e = base + l
                for k in range(D_FEAT // 16):
                    src_v[buf][e, pl.ds(k * 16, 16)] = (
                        src_v[buf][e, pl.ds(k * 16, 16)] * scv)
                src_v[buf][e, pl.ds(D_FEAT, 16)] = lane0 * wvv
            return carry

        lax.fori_loop(0, E_B // 16, _group, 0)

    _fetch_chunk(jnp.int32(0))
    _copy_dst_idx(0, jnp.int32(0))
    _start_gathers(0, jnp.int32(0))

    z16 = jnp.zeros((16,), jnp.float32)

    def _zero_row(r, carry):
        for k in range(D_ACC // 16):
            src_v1[r, pl.ds(k * 16, 16)] = z16
        return carry

    lax.fori_loop(0, ROWS_PER_STAGE, _zero_row, 0)

    def _zero_chunk(j, carry):
        pltpu.sync_copy(
            src_v1.at[pl.ds(0, ROWS_PER_STAGE)],
            acc_sh.at[pl.ds(s * ROWS_PER_TILE + j * ROWS_PER_STAGE,
                            ROWS_PER_STAGE)])
        return carry

    lax.fori_loop(0, ROWS_PER_TILE // ROWS_PER_STAGE, _zero_chunk, 0)
    plsc.subcore_barrier()

    def _pair(p, carry):
        for cur in (0, 1):
            b = 2 * p + cur
            nxt = 1 - cur

            @pl.when(b < nb)
            def _():
                _wait_gathers(cur, b % CHUNK_B)

                @pl.when(b >= 1)
                def _():
                    _wait_scatter(nxt)

                @pl.when(b + 1 < nb)
                def _():
                    @pl.when((b + 1) % CHUNK_B == 0)
                    def _():
                        _fetch_chunk((b + 1) // CHUNK_B)

                    _copy_dst_idx(nxt, (b + 1) % CHUNK_B)
                    _start_gathers(nxt, (b + 1) % CHUNK_B)

                _compute(cur)
                _start_scatter(cur)
        return carry

    lax.fori_loop(0, N_PAIRS, _pair, 0)

    @pl.when(nb % 2 == 1)
    def _():
        _wait_scatter(0)

    @pl.when(nb % 2 == 0)
    def _():
        _wait_scatter(1)

    plsc.subcore_barrier()

    def _out_chunk(j, carry):
        r0 = s * ROWS_PER_TILE + j * ROWS_PER_STAGE
        pltpu.sync_copy(acc_sh.at[pl.ds(r0, ROWS_PER_STAGE)],
                        src_v0.at[pl.ds(0, ROWS_PER_STAGE)])
        pltpu.sync_copy(src_v0.at[pl.ds(0, ROWS_PER_STAGE)],
                        out_hbm.at[c, pl.ds(r0, ROWS_PER_STAGE)])
        return carry

    lax.fori_loop(0, ROWS_PER_TILE // ROWS_PER_STAGE, _out_chunk, 0)


def _sc_edge(aug, dstm, edges):
    mesh = plsc.VectorSubcoreMesh(core_axis_name="c", subcore_axis_name="s",
                                  num_cores=NUM_CORES,
                                  num_subcores=NUM_SUBCORES)
    fn = pl.kernel(
        _sc_body,
        out_type=jax.ShapeDtypeStruct((NUM_CORES, N_NODES, D_ACC),
                                      jnp.float32),
        mesh=mesh,
        scratch_types=[
            pltpu.VMEM((CHUNK_E,), jnp.int32),
            pltpu.VMEM((CHUNK_E,), jnp.int32),
            pltpu.VMEM((E_B,), jnp.int32),
            pltpu.VMEM((E_B,), jnp.int32),
            pltpu.VMEM((E_B, D_ACC), jnp.float32),
            pltpu.VMEM((E_B, D_ACC), jnp.float32),
            pltpu.VMEM((E_B, D_FEAT), jnp.bfloat16),
            pltpu.VMEM((E_B, D_FEAT), jnp.bfloat16),
            pltpu.VMEM((16, 16), jnp.float32),
            pltpu.VMEM_SHARED((N_NODES, D_ACC), jnp.float32),
            pltpu.SemaphoreType.DMA,
            pltpu.SemaphoreType.DMA,
            pltpu.SemaphoreType.DMA,
            pltpu.SemaphoreType.DMA,
            pltpu.SemaphoreType.DMA,
        ],
        compiler_params=pltpu.CompilerParams(use_tc_tiling_on_sc=False,
                                             needs_layout_passes=False),
    )
    return fn(aug, dstm, edges)


def _fin_body(p_ref, o_ref):
    p = p_ref[...]
    tot = p[0] + p[1]
    num = tot[:, :D_FEAT]
    den = tot[:, D_FEAT:D_FEAT + 1]
    o_ref[...] = num / jnp.maximum(den, 1e-20)


def _fin(partial):
    return pl.pallas_call(
        _fin_body,
        out_shape=jax.ShapeDtypeStruct((N_NODES, D_FEAT), jnp.float32),
        grid=(5,),
        in_specs=[pl.BlockSpec((NUM_CORES, N_NODES // 5, D_ACC),
                               lambda i: (0, i, 0))],
        out_specs=pl.BlockSpec((N_NODES // 5, D_FEAT), lambda i: (i, 0)),
    )(partial)


def kernel(feat, edge_index, beta):
    feat = feat.astype(jnp.float32)
    edges = edge_index.astype(jnp.int32)
    edges = jnp.concatenate(
        [edges, jnp.zeros((2, E_PAD - N_EDGES), jnp.int32)], axis=1)
    beta = beta.astype(jnp.float32)
    aug, dstm = _prep(feat, beta)
    partial = _sc_edge(aug, dstm, edges)
    return _fin(partial)
